# baseline (speedup 1.0000x reference)
import functools

import jax
import jax.numpy as jnp
from jax import lax
from jax.experimental import pallas as pl
from jax.experimental.pallas import tpu as pltpu
from jax.experimental.pallas import tpu_sc as plsc

B = 1024
LSEQ = 200
D = 64
NLBL = 100000

NCHUNK = 2
B_CH = B // NCHUNK

NC = 2
NS = 16
NW = NC * NS


def _sc_body(rows_per_w, idx_hbm, mask_hbm, emb_hbm, out_hbm,
             idx_v, mask_v, rows_v, sums_v, sem):
    wid = lax.axis_index("s") * NC + lax.axis_index("c")
    base = wid * rows_per_w
    pltpu.sync_copy(idx_hbm.at[pl.ds(base, rows_per_w)], idx_v)
    pltpu.sync_copy(mask_hbm.at[pl.ds(base, rows_per_w)], mask_v)
    for b in range(rows_per_w):
        pltpu.async_copy(emb_hbm.at[idx_v.at[b]], rows_v, sem).wait()

        def body(l, accs, b=b):
            m = plsc.load_gather(
                mask_v,
                [jnp.full((16,), b, jnp.int32), jnp.full((16,), l, jnp.int32)],
            )
            return tuple(
                accs[j] + rows_v[l, pl.ds(j * 16, 16)] * m for j in range(4)
            )

        accs = lax.fori_loop(
            0, LSEQ, body,
            tuple(jnp.zeros((16,), jnp.float32) for _ in range(4)),
        )
        for j in range(4):
            sums_v[b, pl.ds(j * 16, 16)] = accs[j]
    pltpu.sync_copy(sums_v, out_hbm.at[pl.ds(base, rows_per_w)])


@functools.cache
def _sc_masked_sum_fn(rows):
    rows_per_w = rows // NW
    mesh = plsc.VectorSubcoreMesh(
        core_axis_name="c", subcore_axis_name="s", num_cores=NC, num_subcores=NS
    )
    return pl.kernel(
        functools.partial(_sc_body, rows_per_w),
        out_type=jax.ShapeDtypeStruct((rows, D), jnp.float32),
        mesh=mesh,
        scratch_types=[
            pltpu.VMEM((rows_per_w, LSEQ), jnp.int32),
            pltpu.VMEM((rows_per_w, LSEQ), jnp.float32),
            pltpu.VMEM((LSEQ, D), jnp.float32),
            pltpu.VMEM((rows_per_w, D), jnp.float32),
            pltpu.SemaphoreType.DMA,
        ],
        compiler_params=pltpu.CompilerParams(
            use_tc_tiling_on_sc=False, needs_layout_passes=False
        ),
    )


BLK_N = 2048
_GRID_N = (NLBL + BLK_N - 1) // BLK_N


def _tc_body(has_buf, sums_ref, mask_ref, w_ref, *rest):
    if has_buf:
        _buf_ref, out_ref, normed_ref = rest
    else:
        out_ref, normed_ref = rest

    @pl.when(pl.program_id(0) == 0)
    def _():
        msum = jnp.sum(mask_ref[...], axis=1, keepdims=True)
        v = sums_ref[...] / jnp.maximum(msum, 1.0)
        nrm = jnp.sqrt(jnp.sum(v * v, axis=1, keepdims=True))
        normed_ref[...] = v / jnp.maximum(nrm, 1e-4)

    out_ref[...] = lax.dot_general(
        w_ref[...], normed_ref[...],
        ((( 0,), (1,)), ((), ())),
        preferred_element_type=jnp.float32,
        precision=lax.Precision.DEFAULT,
    )


def _tc_matmul(chunk, sums, mask, w, buf):
    in_specs = [
        pl.BlockSpec((B_CH, D), lambda i: (0, 0)),
        pl.BlockSpec((B_CH, LSEQ), lambda i: (0, 0)),
        pl.BlockSpec((D, BLK_N), lambda i: (0, i)),
    ]
    args = [sums, mask, w]
    aliases = {}
    if buf is not None:
        in_specs.append(pl.BlockSpec(memory_space=pl.ANY))
        args.append(buf)
        aliases = {3: 0}
    return pl.pallas_call(
        functools.partial(_tc_body, buf is not None),
        grid=(_GRID_N,),
        in_specs=in_specs,
        out_specs=pl.BlockSpec((BLK_N, B_CH), lambda i, c=chunk: (i, c)),
        out_shape=jax.ShapeDtypeStruct((NLBL, B), jnp.float32),
        scratch_shapes=[pltpu.VMEM((B_CH, D), jnp.float32)],
        input_output_aliases=aliases,
    )(*args)


def kernel(indices, mask, embedding, kernel):
    sc = _sc_masked_sum_fn(B_CH)
    sums = [
        sc(indices[c * B_CH:(c + 1) * B_CH], mask[c * B_CH:(c + 1) * B_CH],
           embedding)
        for c in range(NCHUNK)
    ]
    out = None
    for c in range(NCHUNK):
        out = _tc_matmul(c, sums[c], mask[c * B_CH:(c + 1) * B_CH], kernel, out)
    return out.T


# speedup vs baseline: 1.0832x; 1.0832x over previous
import functools

import jax
import jax.numpy as jnp
from jax import lax
from jax.experimental import pallas as pl
from jax.experimental.pallas import tpu as pltpu
from jax.experimental.pallas import tpu_sc as plsc

B = 1024
LSEQ = 200
D = 64
NLBL = 100000

NCHUNK = 2
B_CH = B // NCHUNK

NC = 2
NS = 16
NW = NC * NS


def _sc_body(rows_per_w, idx_hbm, mask_hbm, emb_hbm, out_hbm,
             idx_v, mask_v, rows_v, sums_v, sem):
    wid = lax.axis_index("s") * NC + lax.axis_index("c")
    base = wid * rows_per_w
    pltpu.sync_copy(idx_hbm.at[pl.ds(base, rows_per_w)], idx_v)
    pltpu.sync_copy(mask_hbm.at[pl.ds(base, rows_per_w)], mask_v)
    for b in range(rows_per_w):
        pltpu.async_copy(emb_hbm.at[idx_v.at[b]], rows_v, sem).wait()

        def body(l, accs, b=b):
            m = plsc.load_gather(
                mask_v,
                [jnp.full((16,), b, jnp.int32), jnp.full((16,), l, jnp.int32)],
            )
            return tuple(
                accs[j] + rows_v[l, pl.ds(j * 16, 16)] * m for j in range(4)
            )

        accs = lax.fori_loop(
            0, LSEQ, body,
            tuple(jnp.zeros((16,), jnp.float32) for _ in range(4)),
        )
        for j in range(4):
            sums_v[b, pl.ds(j * 16, 16)] = accs[j]
    pltpu.sync_copy(sums_v, out_hbm.at[pl.ds(base, rows_per_w)])


@functools.cache
def _sc_masked_sum_fn(rows):
    rows_per_w = rows // NW
    mesh = plsc.VectorSubcoreMesh(
        core_axis_name="c", subcore_axis_name="s", num_cores=NC, num_subcores=NS
    )
    return pl.kernel(
        functools.partial(_sc_body, rows_per_w),
        out_type=jax.ShapeDtypeStruct((rows, D), jnp.float32),
        mesh=mesh,
        scratch_types=[
            pltpu.VMEM((rows_per_w, LSEQ), jnp.int32),
            pltpu.VMEM((rows_per_w, LSEQ), jnp.float32),
            pltpu.VMEM((LSEQ, D), jnp.float32),
            pltpu.VMEM((rows_per_w, D), jnp.float32),
            pltpu.SemaphoreType.DMA,
        ],
        compiler_params=pltpu.CompilerParams(
            use_tc_tiling_on_sc=False, needs_layout_passes=False
        ),
    )


BLK_N = 4096
_GRID_N = (NLBL + BLK_N - 1) // BLK_N


def _tc_body(has_buf, sums_ref, mask_ref, w_ref, *rest):
    if has_buf:
        _buf_ref, out_ref, normed_ref = rest
    else:
        out_ref, normed_ref = rest

    @pl.when(pl.program_id(0) == 0)
    def _():
        msum = jnp.sum(mask_ref[...], axis=1, keepdims=True)
        v = sums_ref[...] / jnp.maximum(msum, 1.0)
        nrm = jnp.sqrt(jnp.sum(v * v, axis=1, keepdims=True))
        normed_ref[...] = v / jnp.maximum(nrm, 1e-4)

    out_ref[...] = lax.dot_general(
        w_ref[...], normed_ref[...],
        ((( 0,), (1,)), ((), ())),
        preferred_element_type=jnp.float32,
        precision=lax.Precision.DEFAULT,
    )


def _tc_matmul(chunk, sums, mask, w, buf):
    in_specs = [
        pl.BlockSpec((B_CH, D), lambda i: (0, 0)),
        pl.BlockSpec((B_CH, LSEQ), lambda i: (0, 0)),
        pl.BlockSpec((D, BLK_N), lambda i: (0, i)),
    ]
    args = [sums, mask, w]
    aliases = {}
    if buf is not None:
        in_specs.append(pl.BlockSpec(memory_space=pl.ANY))
        args.append(buf)
        aliases = {3: 0}
    return pl.pallas_call(
        functools.partial(_tc_body, buf is not None),
        grid=(_GRID_N,),
        in_specs=in_specs,
        out_specs=pl.BlockSpec((BLK_N, B_CH), lambda i, c=chunk: (i, c)),
        out_shape=jax.ShapeDtypeStruct((NLBL, B), jnp.float32),
        scratch_shapes=[pltpu.VMEM((B_CH, D), jnp.float32)],
        input_output_aliases=aliases,
    )(*args)


def kernel(indices, mask, embedding, kernel):
    sc = _sc_masked_sum_fn(B_CH)
    sums = [
        sc(indices[c * B_CH:(c + 1) * B_CH], mask[c * B_CH:(c + 1) * B_CH],
           embedding)
        for c in range(NCHUNK)
    ]
    out = None
    for c in range(NCHUNK):
        out = _tc_matmul(c, sums[c], mask[c * B_CH:(c + 1) * B_CH], kernel, out)
    return out.T


# speedup vs baseline: 1.1623x; 1.0731x over previous
import functools

import jax
import jax.numpy as jnp
from jax import lax
from jax.experimental import pallas as pl
from jax.experimental.pallas import tpu as pltpu
from jax.experimental.pallas import tpu_sc as plsc

B = 1024
LSEQ = 200
D = 64
NLBL = 100000

NCHUNK = 1
B_CH = B // NCHUNK

NC = 2
NS = 16
NW = NC * NS


def _sc_body(rows_per_w, idx_hbm, mask_hbm, emb_hbm, out_hbm,
             idx_v, mask_v, rows_v, sums_v, sem):
    wid = lax.axis_index("s") * NC + lax.axis_index("c")
    base = wid * rows_per_w
    pltpu.sync_copy(idx_hbm.at[pl.ds(base, rows_per_w)], idx_v)
    pltpu.sync_copy(mask_hbm.at[pl.ds(base, rows_per_w)], mask_v)
    for b in range(rows_per_w):
        pltpu.async_copy(emb_hbm.at[idx_v.at[b]], rows_v, sem).wait()

        def body(l, accs, b=b):
            m = plsc.load_gather(
                mask_v,
                [jnp.full((16,), b, jnp.int32), jnp.full((16,), l, jnp.int32)],
            )
            return tuple(
                accs[j] + rows_v[l, pl.ds(j * 16, 16)] * m for j in range(4)
            )

        accs = lax.fori_loop(
            0, LSEQ, body,
            tuple(jnp.zeros((16,), jnp.float32) for _ in range(4)),
        )
        for j in range(4):
            sums_v[b, pl.ds(j * 16, 16)] = accs[j]
    pltpu.sync_copy(sums_v, out_hbm.at[pl.ds(base, rows_per_w)])


@functools.cache
def _sc_masked_sum_fn(rows):
    rows_per_w = rows // NW
    mesh = plsc.VectorSubcoreMesh(
        core_axis_name="c", subcore_axis_name="s", num_cores=NC, num_subcores=NS
    )
    return pl.kernel(
        functools.partial(_sc_body, rows_per_w),
        out_type=jax.ShapeDtypeStruct((rows, D), jnp.float32),
        mesh=mesh,
        scratch_types=[
            pltpu.VMEM((rows_per_w, LSEQ), jnp.int32),
            pltpu.VMEM((rows_per_w, LSEQ), jnp.float32),
            pltpu.VMEM((LSEQ, D), jnp.float32),
            pltpu.VMEM((rows_per_w, D), jnp.float32),
            pltpu.SemaphoreType.DMA,
        ],
        compiler_params=pltpu.CompilerParams(
            use_tc_tiling_on_sc=False, needs_layout_passes=False
        ),
    )


BLK_N = 4096
_GRID_N = (NLBL + BLK_N - 1) // BLK_N


def _tc_body(has_buf, sums_ref, mask_ref, w_ref, *rest):
    if has_buf:
        _buf_ref, out_ref, normed_ref = rest
    else:
        out_ref, normed_ref = rest

    @pl.when(pl.program_id(0) == 0)
    def _():
        msum = jnp.sum(mask_ref[...], axis=1, keepdims=True)
        v = sums_ref[...] / jnp.maximum(msum, 1.0)
        nrm = jnp.sqrt(jnp.sum(v * v, axis=1, keepdims=True))
        normed_ref[...] = v / jnp.maximum(nrm, 1e-4)

    out_ref[...] = lax.dot_general(
        w_ref[...], normed_ref[...],
        ((( 0,), (1,)), ((), ())),
        preferred_element_type=jnp.float32,
        precision=lax.Precision.DEFAULT,
    )


def _tc_matmul(chunk, sums, mask, w, buf):
    in_specs = [
        pl.BlockSpec((B_CH, D), lambda i: (0, 0)),
        pl.BlockSpec((B_CH, LSEQ), lambda i: (0, 0)),
        pl.BlockSpec((D, BLK_N), lambda i: (0, i)),
    ]
    args = [sums, mask, w]
    aliases = {}
    if buf is not None:
        in_specs.append(pl.BlockSpec(memory_space=pl.ANY))
        args.append(buf)
        aliases = {3: 0}
    return pl.pallas_call(
        functools.partial(_tc_body, buf is not None),
        grid=(_GRID_N,),
        in_specs=in_specs,
        out_specs=pl.BlockSpec((BLK_N, B_CH), lambda i, c=chunk: (i, c)),
        out_shape=jax.ShapeDtypeStruct((NLBL, B), jnp.float32),
        scratch_shapes=[pltpu.VMEM((B_CH, D), jnp.float32)],
        input_output_aliases=aliases,
    )(*args)


def kernel(indices, mask, embedding, kernel):
    sc = _sc_masked_sum_fn(B_CH)
    sums = [
        sc(indices[c * B_CH:(c + 1) * B_CH], mask[c * B_CH:(c + 1) * B_CH],
           embedding)
        for c in range(NCHUNK)
    ]
    out = None
    for c in range(NCHUNK):
        out = _tc_matmul(c, sums[c], mask[c * B_CH:(c + 1) * B_CH], kernel, out)
    return out.T


# speedup vs baseline: 1.2846x; 1.1052x over previous
import functools

import jax
import jax.numpy as jnp
from jax import lax
from jax.experimental import pallas as pl
from jax.experimental.pallas import tpu as pltpu
from jax.experimental.pallas import tpu_sc as plsc

B = 1024
LSEQ = 200
D = 64
NLBL = 100000

NCHUNK = 1
B_CH = B // NCHUNK

NC = 2
NS = 16
NW = NC * NS


def _sc_body(rows_per_w, idx_hbm, mask_hbm, emb_hbm, out_hbm,
             idx_v, mask_v, rows_v0, rows_v1, sums_v, sem0, sem1):
    wid = lax.axis_index("s") * NC + lax.axis_index("c")
    base = wid * rows_per_w
    pltpu.sync_copy(idx_hbm.at[pl.ds(base, rows_per_w)], idx_v)
    pltpu.sync_copy(mask_hbm.at[pl.ds(base, rows_per_w)], mask_v)
    bufs = (rows_v0, rows_v1)
    sems = (sem0, sem1)
    pending = pltpu.async_copy(emb_hbm.at[idx_v.at[0]], bufs[0], sems[0])
    for b in range(rows_per_w):
        if b + 1 < rows_per_w:
            nxt = pltpu.async_copy(
                emb_hbm.at[idx_v.at[b + 1]], bufs[(b + 1) % 2], sems[(b + 1) % 2]
            )
        pending.wait()
        rows_v = bufs[b % 2]

        def body(l, accs, b=b, rows_v=rows_v):
            m = plsc.load_gather(
                mask_v,
                [jnp.full((16,), b, jnp.int32), jnp.full((16,), l, jnp.int32)],
            )
            return tuple(
                accs[j] + rows_v[l, pl.ds(j * 16, 16)] * m for j in range(4)
            )

        accs = lax.fori_loop(
            0, LSEQ, body,
            tuple(jnp.zeros((16,), jnp.float32) for _ in range(4)),
            unroll=8,
        )
        for j in range(4):
            sums_v[b, pl.ds(j * 16, 16)] = accs[j]
        if b + 1 < rows_per_w:
            pending = nxt
    pltpu.sync_copy(sums_v, out_hbm.at[pl.ds(base, rows_per_w)])


@functools.cache
def _sc_masked_sum_fn(rows):
    rows_per_w = rows // NW
    mesh = plsc.VectorSubcoreMesh(
        core_axis_name="c", subcore_axis_name="s", num_cores=NC, num_subcores=NS
    )
    return pl.kernel(
        functools.partial(_sc_body, rows_per_w),
        out_type=jax.ShapeDtypeStruct((rows, D), jnp.float32),
        mesh=mesh,
        scratch_types=[
            pltpu.VMEM((rows_per_w, LSEQ), jnp.int32),
            pltpu.VMEM((rows_per_w, LSEQ), jnp.float32),
            pltpu.VMEM((LSEQ, D), jnp.float32),
            pltpu.VMEM((LSEQ, D), jnp.float32),
            pltpu.VMEM((rows_per_w, D), jnp.float32),
            pltpu.SemaphoreType.DMA,
            pltpu.SemaphoreType.DMA,
        ],
        compiler_params=pltpu.CompilerParams(
            use_tc_tiling_on_sc=False, needs_layout_passes=False
        ),
    )


BLK_N = 4096
_GRID_N = (NLBL + BLK_N - 1) // BLK_N


def _tc_body(has_buf, sums_ref, mask_ref, w_ref, *rest):
    if has_buf:
        _buf_ref, out_ref, normed_ref = rest
    else:
        out_ref, normed_ref = rest

    @pl.when(pl.program_id(0) == 0)
    def _():
        msum = jnp.sum(mask_ref[...], axis=1, keepdims=True)
        v = sums_ref[...] / jnp.maximum(msum, 1.0)
        nrm = jnp.sqrt(jnp.sum(v * v, axis=1, keepdims=True))
        normed_ref[...] = v / jnp.maximum(nrm, 1e-4)

    out_ref[...] = lax.dot_general(
        w_ref[...], normed_ref[...],
        ((( 0,), (1,)), ((), ())),
        preferred_element_type=jnp.float32,
        precision=lax.Precision.DEFAULT,
    )


def _tc_matmul(chunk, sums, mask, w, buf):
    in_specs = [
        pl.BlockSpec((B_CH, D), lambda i: (0, 0)),
        pl.BlockSpec((B_CH, LSEQ), lambda i: (0, 0)),
        pl.BlockSpec((D, BLK_N), lambda i: (0, i)),
    ]
    args = [sums, mask, w]
    aliases = {}
    if buf is not None:
        in_specs.append(pl.BlockSpec(memory_space=pl.ANY))
        args.append(buf)
        aliases = {3: 0}
    return pl.pallas_call(
        functools.partial(_tc_body, buf is not None),
        grid=(_GRID_N,),
        in_specs=in_specs,
        out_specs=pl.BlockSpec((BLK_N, B_CH), lambda i, c=chunk: (i, c)),
        out_shape=jax.ShapeDtypeStruct((NLBL, B), jnp.float32),
        scratch_shapes=[pltpu.VMEM((B_CH, D), jnp.float32)],
        input_output_aliases=aliases,
    )(*args)


def kernel(indices, mask, embedding, kernel):
    sc = _sc_masked_sum_fn(B_CH)
    sums = [
        sc(indices[c * B_CH:(c + 1) * B_CH], mask[c * B_CH:(c + 1) * B_CH],
           embedding)
        for c in range(NCHUNK)
    ]
    out = None
    for c in range(NCHUNK):
        out = _tc_matmul(c, sums[c], mask[c * B_CH:(c + 1) * B_CH], kernel, out)
    return out.T
